# Initial kernel scaffold; baseline (speedup 1.0000x reference)
import jax
import jax.numpy as jnp
from jax.experimental import pallas as pl


def kernel(z_e, embedding_weight):
    raise NotImplementedError("write your pallas kernel here")


# speedup vs baseline: 1.0159x; 1.0159x over previous
import jax
import jax.numpy as jnp
from jax.experimental import pallas as pl

NUM_EMBEDDINGS = 1024
EMBEDDING_DIM = 64
ROWS_PER_BLOCK = 2048


def _vq_block_kernel(z_ref, e_ref, zq_ref, idx_ref):
    z = z_ref[...]
    e = e_ref[...]
    zsq = jnp.sum(z * z, axis=1, keepdims=True)
    esq = jnp.sum(e * e, axis=1)
    mm = jax.lax.dot_general(
        z, e, (((1,), (1,)), ((), ())),
        preferred_element_type=jnp.float32)
    dist = (zsq + esq[None, :]) - 2.0 * mm
    mins = jnp.min(dist, axis=1, keepdims=True)
    iota = jax.lax.broadcasted_iota(jnp.int32, dist.shape, 1)
    idx = jnp.min(jnp.where(dist == mins, iota, NUM_EMBEDDINGS), axis=1)
    idx_ref[...] = idx
    onehot = (iota == idx[:, None]).astype(jnp.float32)
    zq_ref[...] = jax.lax.dot_general(
        onehot, e, (((1,), (0,)), ((), ())),
        precision=jax.lax.Precision.HIGHEST,
        preferred_element_type=jnp.float32)


def kernel(z_e, embedding_weight):
    b, c, h, w = z_e.shape
    n = b * h * w
    z_flat = jnp.transpose(z_e, (0, 2, 3, 1)).reshape(n, c)
    nblk = n // ROWS_PER_BLOCK
    zq_flat, idx = pl.pallas_call(
        _vq_block_kernel,
        grid=(nblk,),
        in_specs=[
            pl.BlockSpec((ROWS_PER_BLOCK, c), lambda i: (i, 0)),
            pl.BlockSpec((NUM_EMBEDDINGS, c), lambda i: (0, 0)),
        ],
        out_specs=[
            pl.BlockSpec((ROWS_PER_BLOCK, c), lambda i: (i, 0)),
            pl.BlockSpec((ROWS_PER_BLOCK,), lambda i: (i,)),
        ],
        out_shape=[
            jax.ShapeDtypeStruct((n, c), jnp.float32),
            jax.ShapeDtypeStruct((n,), jnp.int32),
        ],
    )(z_flat, embedding_weight)
    return zq_flat.reshape(z_e.shape), idx


# speedup vs baseline: 1.7210x; 1.6940x over previous
import jax
import jax.numpy as jnp
from jax.experimental import pallas as pl

NUM_EMBEDDINGS = 1024
EMBEDDING_DIM = 64
ROWS_PER_BLOCK = 2048


def _vq_block_kernel(z_ref, e_ref, zq_ref, idx_ref):
    z = z_ref[...]
    e = e_ref[...]
    zsq = jnp.sum(z * z, axis=1, keepdims=True)
    esq = jnp.sum(e * e, axis=1)
    mm2 = jax.lax.dot_general(
        z * (-2.0), e, (((1,), (1,)), ((), ())),
        preferred_element_type=jnp.float32)
    dist = (zsq + esq[None, :]) + mm2
    idx = jnp.argmin(dist, axis=1)
    idx_ref[...] = idx
    iota = jax.lax.broadcasted_iota(jnp.int32, dist.shape, 1)
    onehot = (iota == idx[:, None]).astype(jnp.float32)
    zq_ref[...] = jax.lax.dot_general(
        onehot, e, (((1,), (0,)), ((), ())),
        preferred_element_type=jnp.float32)


def kernel(z_e, embedding_weight):
    b, c, h, w = z_e.shape
    n = b * h * w
    z_flat = jnp.transpose(z_e, (0, 2, 3, 1)).reshape(n, c)
    nblk = n // ROWS_PER_BLOCK
    zq_flat, idx = pl.pallas_call(
        _vq_block_kernel,
        grid=(nblk,),
        in_specs=[
            pl.BlockSpec((ROWS_PER_BLOCK, c), lambda i: (i, 0)),
            pl.BlockSpec((NUM_EMBEDDINGS, c), lambda i: (0, 0)),
        ],
        out_specs=[
            pl.BlockSpec((ROWS_PER_BLOCK, c), lambda i: (i, 0)),
            pl.BlockSpec((ROWS_PER_BLOCK,), lambda i: (i,)),
        ],
        out_shape=[
            jax.ShapeDtypeStruct((n, c), jnp.float32),
            jax.ShapeDtypeStruct((n,), jnp.int32),
        ],
    )(z_flat, embedding_weight)
    return zq_flat.reshape(z_e.shape), idx
